# baseline (speedup 1.0000x reference)
import functools

import jax
import jax.numpy as jnp
from jax.experimental import pallas as pl
from jax.experimental.pallas import tpu as pltpu

N_ROWS_PER_BLOCK = 1000


def _fused_embed_kernel(x_ref, w_ref, b_ref, src_ref, rx_ref):
    y = jnp.dot(x_ref[...], w_ref[...], preferred_element_type=jnp.float32)
    y = y + b_ref[...]
    src_ref[...] = y[:, :32]
    rx_ref[...] = y[:, 32:]


@jax.jit
def kernel(x, edge_index, W_src, b_src, W_rx, b_rx):
    del edge_index
    n = x.shape[0]
    w = jnp.concatenate([W_src.T, W_rx.T], axis=1)
    b = jnp.concatenate([b_src, b_rx])[None, :]

    grid = n // N_ROWS_PER_BLOCK
    src, rx = pl.pallas_call(
        _fused_embed_kernel,
        grid=(grid,),
        in_specs=[
            pl.BlockSpec((N_ROWS_PER_BLOCK, x.shape[1]), lambda i: (i, 0)),
            pl.BlockSpec((x.shape[1], 64), lambda i: (0, 0)),
            pl.BlockSpec((1, 64), lambda i: (0, 0)),
        ],
        out_specs=[
            pl.BlockSpec((N_ROWS_PER_BLOCK, 32), lambda i: (i, 0)),
            pl.BlockSpec((N_ROWS_PER_BLOCK, 32), lambda i: (i, 0)),
        ],
        out_shape=[
            jax.ShapeDtypeStruct((n, 32), jnp.float32),
            jax.ShapeDtypeStruct((n, 32), jnp.float32),
        ],
        compiler_params=pltpu.CompilerParams(
            dimension_semantics=("parallel",),
        ),
    )(x, w, b)
    return (src, rx)


# speedup vs baseline: 1.1646x; 1.1646x over previous
import jax
import jax.numpy as jnp
from jax import lax
from jax.experimental import pallas as pl
from jax.experimental.pallas import tpu as pltpu

N_ROWS_PER_BLOCK = 1000

_DNUMS = (((1,), (1,)), ((), ()))


def _fused_embed_kernel(x_ref, ws_ref, bs_ref, wr_ref, br_ref, src_ref, rx_ref):
    x = x_ref[...]
    src_ref[...] = lax.dot_general(
        x, ws_ref[...], _DNUMS, preferred_element_type=jnp.float32
    ) + bs_ref[...]
    rx_ref[...] = lax.dot_general(
        x, wr_ref[...], _DNUMS, preferred_element_type=jnp.float32
    ) + br_ref[...]


@jax.jit
def kernel(x, edge_index, W_src, b_src, W_rx, b_rx):
    del edge_index
    n, k = x.shape
    grid = n // N_ROWS_PER_BLOCK
    src, rx = pl.pallas_call(
        _fused_embed_kernel,
        grid=(grid,),
        in_specs=[
            pl.BlockSpec((N_ROWS_PER_BLOCK, k), lambda i: (i, 0)),
            pl.BlockSpec((32, k), lambda i: (0, 0)),
            pl.BlockSpec((32,), lambda i: (0,)),
            pl.BlockSpec((32, k), lambda i: (0, 0)),
            pl.BlockSpec((32,), lambda i: (0,)),
        ],
        out_specs=[
            pl.BlockSpec((N_ROWS_PER_BLOCK, 32), lambda i: (i, 0)),
            pl.BlockSpec((N_ROWS_PER_BLOCK, 32), lambda i: (i, 0)),
        ],
        out_shape=[
            jax.ShapeDtypeStruct((n, 32), jnp.float32),
            jax.ShapeDtypeStruct((n, 32), jnp.float32),
        ],
        compiler_params=pltpu.CompilerParams(
            dimension_semantics=("arbitrary",),
        ),
    )(x, W_src, b_src, W_rx, b_rx)
    return (src, rx)
